# baseline (speedup 1.0000x reference)
import functools

import jax
import jax.numpy as jnp
from jax import lax
from jax.experimental import pallas as pl
from jax.experimental.pallas import tpu as pltpu
from jax.experimental.pallas import tpu_sc as plsc

H = 384
W = 384
C = 192
HW = H * W
K = 30
NEG = float("-inf")
CONV_BN = 12288


CONV_BH = 32


def _conv_body(w_ref, b_ref, f_ref, o_ref):
    acc = w_ref[0] * f_ref[0, 0]
    for c in range(1, C):
        acc = acc + w_ref[c] * f_ref[0, c]
    o_ref[...] = jnp.clip(jax.nn.sigmoid(acc + b_ref[0]), 0.0001, 1.0 - 0.0001)


def _conv_center(w, b, f4):
    return pl.pallas_call(
        _conv_body,
        grid=(H // CONV_BH,),
        in_specs=[
            pl.BlockSpec(memory_space=pltpu.SMEM),
            pl.BlockSpec(memory_space=pltpu.SMEM),
            pl.BlockSpec((1, C, CONV_BH, W), lambda i: (0, 0, i, 0)),
        ],
        out_specs=pl.BlockSpec((CONV_BH, W), lambda i: (i, 0)),
        out_shape=jax.ShapeDtypeStruct((H, W), jnp.float32),
    )(w, b, f4)


def _shift_rows(x, dy, fill):
    if dy == 0:
        return x
    blk = jnp.full((abs(dy), x.shape[1]), fill, x.dtype)
    if dy > 0:
        return jnp.concatenate([x[dy:, :], blk], axis=0)
    return jnp.concatenate([blk, x[:dy, :]], axis=0)


def _shift_cols(x, dx, fill):
    if dx == 0:
        return x
    blk = jnp.full((x.shape[0], abs(dx)), fill, x.dtype)
    if dx > 0:
        return jnp.concatenate([x[:, dx:], blk], axis=1)
    return jnp.concatenate([blk, x[:, :dx]], axis=1)


def _select_body(c_ref, scores_ref, idx_ref, ys_ref, xs_ref, m_ref, rmax_ref):
    c = c_ref[...]
    rowsum = c + _shift_cols(c, -1, 0.0) + _shift_cols(c, 1, 0.0)
    s = rowsum + _shift_rows(rowsum, -1, 0.0) + _shift_rows(rowsum, 1, 0.0)
    c2 = (c + s / 9.0) / 2.0
    rm = c2
    for dx in (-3, -2, -1, 1, 2, 3):
        rm = jnp.maximum(rm, _shift_cols(c2, dx, NEG))
    mm = rm
    for dy in (-3, -2, -1, 1, 2, 3):
        mm = jnp.maximum(mm, _shift_rows(rm, dy, NEG))
    masked = jnp.where(mm == c2, c2, 0.0)
    m_ref[...] = masked
    rmax_ref[...] = jnp.max(masked, axis=1, keepdims=True)

    lane_iota = lax.broadcasted_iota(jnp.int32, (1, W), 1)
    row_iota = lax.broadcasted_iota(jnp.int32, (H, 1), 0)
    ch_iota = lax.broadcasted_iota(jnp.int32, (1, C), 1)

    def body(i, carry):
        rmax = rmax_ref[...]
        gmax = jnp.max(rmax)
        h = jnp.min(jnp.where(rmax == gmax, row_iota, H))
        row = m_ref[pl.ds(h, 1), :]
        wj = jnp.min(jnp.where(row == gmax, lane_iota, W))
        newrow = jnp.where(lane_iota == wj, NEG, row)
        m_ref[pl.ds(h, 1), :] = newrow
        rmax_ref[pl.ds(h, 1), :] = jnp.max(newrow, axis=1, keepdims=True)
        scores_ref[i] = gmax
        idx_ref[pl.ds(i, 1), :] = ch_iota * HW + (h * W + wj)
        ys_ref[i] = h
        xs_ref[i] = wj
        return carry

    lax.fori_loop(0, K, body, 0)


def _select_topk(center):
    return pl.pallas_call(
        _select_body,
        in_specs=[pl.BlockSpec((H, W), lambda: (0, 0))],
        out_specs=[
            pl.BlockSpec(memory_space=pltpu.SMEM),
            pl.BlockSpec((32, C), lambda: (0, 0)),
            pl.BlockSpec(memory_space=pltpu.SMEM),
            pl.BlockSpec(memory_space=pltpu.SMEM),
        ],
        out_shape=[
            jax.ShapeDtypeStruct((32,), jnp.float32),
            jax.ShapeDtypeStruct((32, C), jnp.int32),
            jax.ShapeDtypeStruct((32,), jnp.int32),
            jax.ShapeDtypeStruct((32,), jnp.int32),
        ],
        scratch_shapes=[
            pltpu.VMEM((H, W), jnp.float32),
            pltpu.VMEM((H, 1), jnp.float32),
        ],
    )(center)


def _gather_params(f_flat, idx):

    @functools.partial(
        pl.kernel,
        out_type=jax.ShapeDtypeStruct((K * C,), jnp.float32),
        mesh=plsc.VectorSubcoreMesh(core_axis_name="c", subcore_axis_name="s"),
        scratch_types=[
            pltpu.VMEM((C,), jnp.int32),
            pltpu.VMEM((C,), jnp.float32),
            pltpu.SemaphoreType.DMA,
        ],
    )
    def k(table, idx_hbm, out_hbm, idx_v, out_v, sem):
        wid = lax.axis_index("s") * 2 + lax.axis_index("c")

        @pl.when(wid < K)
        def _():
            pltpu.sync_copy(idx_hbm.at[pl.ds(wid * C, C)], idx_v)
            copies = []
            for j in range(C // 16):
                iv = idx_v[pl.ds(16 * j, 16)]
                copies.append(
                    pltpu.async_copy(
                        table.at[iv], out_v.at[pl.ds(16 * j, 16)], sem
                    )
                )
            for cp in copies:
                cp.wait()
            pltpu.sync_copy(out_v, out_hbm.at[pl.ds(wid * C, C)])

    return k(f_flat, idx)


def kernel(features, conv_w, conv_b):
    w = conv_w[-1]
    b = conv_b[-1:]
    center = _conv_center(w, b, features)
    scores32, idx32, ys32, xs32 = _select_topk(center)
    params = _gather_params(features.reshape(-1), idx32.reshape(-1))
    instance_coord = jnp.stack([ys32[:K], xs32[:K]], axis=1)
    instance_imgid = jnp.zeros((K,), jnp.int32)
    instance_param = params.reshape(K, C)
    scores = scores32[:K]
    return (instance_coord, instance_imgid, instance_param, scores)


# speedup vs baseline: 1.1280x; 1.1280x over previous
import jax
import jax.numpy as jnp
from jax import lax
from jax.experimental import pallas as pl
from jax.experimental.pallas import tpu as pltpu

H = 384
W = 384
C = 192
HW = H * W
K = 30
NEG = float("-inf")
CONV_BN = 12288


def _conv_body(w_ref, b_ref, f_ref, o_ref):
    x = jnp.dot(w_ref[...], f_ref[...], preferred_element_type=jnp.float32)
    x = x + b_ref[0, 0]
    o_ref[...] = jnp.clip(jax.nn.sigmoid(x), 0.0001, 1.0 - 0.0001)


def _conv_center(w, b, f2):
    return pl.pallas_call(
        _conv_body,
        grid=(HW // CONV_BN,),
        in_specs=[
            pl.BlockSpec((1, C), lambda i: (0, 0)),
            pl.BlockSpec(memory_space=pltpu.SMEM),
            pl.BlockSpec((C, CONV_BN), lambda i: (0, i)),
        ],
        out_specs=pl.BlockSpec((1, CONV_BN), lambda i: (0, i)),
        out_shape=jax.ShapeDtypeStruct((1, HW), jnp.float32),
    )(w, b, f2)


def _shift_rows(x, dy, fill):
    if dy == 0:
        return x
    blk = jnp.full((abs(dy), x.shape[1]), fill, x.dtype)
    if dy > 0:
        return jnp.concatenate([x[dy:, :], blk], axis=0)
    return jnp.concatenate([blk, x[:dy, :]], axis=0)


def _shift_cols(x, dx, fill):
    if dx == 0:
        return x
    blk = jnp.full((x.shape[0], abs(dx)), fill, x.dtype)
    if dx > 0:
        return jnp.concatenate([x[:, dx:], blk], axis=1)
    return jnp.concatenate([blk, x[:, :dx]], axis=1)


def _select_body(c_ref, f_ref, scores_ref, ys_ref, xs_ref, param_ref,
                 m_ref, rmax_ref, pos_ref, pgwin_ref, pg_ref, sem):
    c = c_ref[...]
    rowsum = c + _shift_cols(c, -1, 0.0) + _shift_cols(c, 1, 0.0)
    s = rowsum + _shift_rows(rowsum, -1, 0.0) + _shift_rows(rowsum, 1, 0.0)
    c2 = (c + s / 9.0) / 2.0
    rm = c2
    for dx in (-3, -2, -1, 1, 2, 3):
        rm = jnp.maximum(rm, _shift_cols(c2, dx, NEG))
    mm = rm
    for dy in (-3, -2, -1, 1, 2, 3):
        mm = jnp.maximum(mm, _shift_rows(rm, dy, NEG))
    masked = jnp.where(mm == c2, c2, 0.0)
    m_ref[...] = masked
    rmax_ref[...] = jnp.max(masked, axis=1, keepdims=True)

    lane_iota = lax.broadcasted_iota(jnp.int32, (1, W), 1)
    row_iota = lax.broadcasted_iota(jnp.int32, (H, 1), 0)

    def body(i, carry):
        rmax = rmax_ref[...]
        gmax = jnp.max(rmax)
        h = jnp.min(jnp.where(rmax == gmax, row_iota, H))
        row = m_ref[pl.ds(h, 1), :]
        wj = jnp.min(jnp.where(row == gmax, lane_iota, W))
        newrow = jnp.where(lane_iota == wj, NEG, row)
        m_ref[pl.ds(h, 1), :] = newrow
        rmax_ref[pl.ds(h, 1), :] = jnp.max(newrow, axis=1, keepdims=True)
        scores_ref[i] = gmax
        pos_ref[i] = h * W + wj
        ys_ref[i] = h
        xs_ref[i] = wj
        return carry

    lax.fori_loop(0, K, body, 0)

    copies = [
        pltpu.make_async_copy(
            f_ref.at[:, pl.ds(pl.multiple_of((pos_ref[p] >> 7) * 128, 128), 128)],
            pgwin_ref.at[:, pl.ds(p * 128, 128)],
            sem,
        )
        for p in range(K)
    ]
    for cp in copies:
        cp.start()
    lane128 = lax.broadcasted_iota(jnp.int32, (1, 128), 1)
    for p, cp in enumerate(copies):
        cp.wait()
        win = pgwin_ref[:, pl.ds(p * 128, 128)]
        onehot = lane128 == (pos_ref[p] & 127)
        pg_ref[:, pl.ds(p, 1)] = jnp.sum(
            jnp.where(onehot, win, 0.0), axis=1, keepdims=True
        )
    eye = jnp.where(
        lax.broadcasted_iota(jnp.int32, (32, 32), 0)
        == lax.broadcasted_iota(jnp.int32, (32, 32), 1),
        1.0,
        0.0,
    )
    param_ref[...] = lax.dot_general(
        eye, pg_ref[...], (((1,), (1,)), ((), ())),
        precision=lax.Precision.HIGHEST,
        preferred_element_type=jnp.float32,
    )


def _select_topk(center, f2):
    return pl.pallas_call(
        _select_body,
        in_specs=[
            pl.BlockSpec((H, W), lambda: (0, 0)),
            pl.BlockSpec(memory_space=pl.ANY),
        ],
        out_specs=[
            pl.BlockSpec(memory_space=pltpu.SMEM),
            pl.BlockSpec(memory_space=pltpu.SMEM),
            pl.BlockSpec(memory_space=pltpu.SMEM),
            pl.BlockSpec((32, C), lambda: (0, 0)),
        ],
        out_shape=[
            jax.ShapeDtypeStruct((32,), jnp.float32),
            jax.ShapeDtypeStruct((32,), jnp.int32),
            jax.ShapeDtypeStruct((32,), jnp.int32),
            jax.ShapeDtypeStruct((32, C), jnp.float32),
        ],
        scratch_shapes=[
            pltpu.VMEM((H, W), jnp.float32),
            pltpu.VMEM((H, 1), jnp.float32),
            pltpu.SMEM((32,), jnp.int32),
            pltpu.VMEM((C, K * 128), jnp.float32),
            pltpu.VMEM((C, 32), jnp.float32),
            pltpu.SemaphoreType.DMA,
        ],
    )(center, f2)


def kernel(features, conv_w, conv_b):
    f2 = features.reshape(C, HW)
    w = conv_w[-1:]
    b = conv_b[-1:].reshape(1, 1)
    center = _conv_center(w, b, f2).reshape(H, W)
    scores32, ys32, xs32, params32 = _select_topk(center, f2)
    instance_coord = jnp.stack([ys32[:K], xs32[:K]], axis=1)
    instance_imgid = jnp.zeros((K,), jnp.int32)
    instance_param = params32[:K]
    scores = scores32[:K]
    return (instance_coord, instance_imgid, instance_param, scores)
